# baseline (speedup 1.0000x reference)
import functools
import jax
import jax.numpy as jnp
from jax import lax
from jax.experimental import pallas as pl
from jax.experimental.pallas import tpu as pltpu
from jax.experimental.pallas import tpu_sc as plsc

HEADS = 4
C = 32
D = HEADS * C
NEG = 0.2
K = 128
L = 16
NC = 2
NS = 16
NW = NC * NS
TRASH = 8


def _proj_body(x_ref, wl_ref, bl_ref, wr_ref, br_ref, xl_ref, xr_ref):
    x = x_ref[...]
    xl_ref[...] = jnp.dot(x, wl_ref[...], preferred_element_type=jnp.float32) + bl_ref[...]
    xr_ref[...] = jnp.dot(x, wr_ref[...], preferred_element_type=jnp.float32) + br_ref[...]


def _xe_body(ea_ref, we_ref, xe_ref):
    xe_ref[...] = jnp.dot(ea_ref[...], we_ref[...], preferred_element_type=jnp.float32)


def _alpha_body(nchunks_tot,
                xl_hbm, xr_hbm, xe_hbm, src_hbm, dst_hbm, ea_hbm, att_hbm,
                aux_out, exr_out,
                attv, src_v, dst_v, ea_v, xlg, xrg, xeg, aux_c,
                abuf, rbuf, ebuf, zbufa, aux_sh, sem1, sem2, sem3):
    n = xl_hbm.shape[0]
    pltpu.sync_copy(att_hbm, attv)
    cid = lax.axis_index("c")
    sid = lax.axis_index("s")
    wid = sid * NC + cid
    iota = lax.iota(jnp.int32, L)
    iota4 = iota * 4
    zero16 = jnp.zeros((L,), jnp.float32)

    zr = zbufa.shape[0]
    for i in range(zr):
        zbufa[i, :] = zero16

    att_r = [attv[pl.ds(j * L, L)] for j in range(D // L)]

    for j in range(1, 4):
        rbuf[pl.ds(j * L, L)] = zero16

    zrows = n // NS
    zrow0 = sid * zrows
    nfull, rem = divmod(zrows, zr)
    for ci in range(nfull):
        pltpu.sync_copy(zbufa, aux_sh.at[pl.ds(zrow0 + ci * zr, zr)])
    if rem:
        pltpu.sync_copy(zbufa.at[pl.ds(0, rem)],
                        aux_sh.at[pl.ds(zrow0 + nfull * zr, rem)])
    plsc.subcore_barrier()

    nch = (nchunks_tot // NW) + jnp.where(wid < (nchunks_tot % NW), 1, 0)

    def chunk_body(i, _):
        chunk = wid + i * NW
        base = chunk * K
        pltpu.sync_copy(src_hbm.at[pl.ds(base, K)], src_v)
        pltpu.sync_copy(dst_hbm.at[pl.ds(base, K)], dst_v)
        pltpu.sync_copy(ea_hbm.at[pl.ds(base * 4, K * 4)],
                        ea_v.at[pl.ds(0, K * 4)])
        cp1 = pltpu.async_copy(xl_hbm.at[src_v], xlg, sem1)
        cp2 = pltpu.async_copy(xr_hbm.at[dst_v], xrg, sem2)
        cp3 = pltpu.async_copy(xe_hbm.at[pl.ds(base, K), :], xeg, sem3)
        cp1.wait()
        cp2.wait()
        cp3.wait()

        def edge_body(e, _):
            xlr = [xlg[e, pl.ds(j * L, L)] for j in range(D // L)]
            xrr = [xrg[e, pl.ds(j * L, L)] for j in range(D // L)]
            xer = [xeg[e, pl.ds(j * L, L)] for j in range(D // L)]
            for h in range(HEADS):
                t = zero16
                for j in (2 * h, 2 * h + 1):
                    u = xlr[j] + xrr[j] + xer[j]
                    lr = jnp.maximum(u, NEG * u)
                    t = t + lr * att_r[j]
                abuf[pl.ds(h * L, L)] = t
            s = zero16
            for rr in range(4):
                s = s + plsc.load_gather(abuf, [iota4 + rr])
            rbuf[pl.ds(0, L)] = s
            w = zero16
            for rr in range(4):
                w = w + plsc.load_gather(rbuf, [iota4 + rr])
            expv = jnp.exp(w)
            eav = plsc.load_gather(ea_v, [4 * e + jnp.maximum(iota - 5, 0)])
            r = jnp.where(iota < 5, expv, jnp.where(iota < 8, eav, zero16))
            aux_c[e, :] = r
            return _

        lax.fori_loop(0, K, edge_body, None, unroll=4)

        pltpu.sync_copy(aux_c, exr_out.at[pl.ds(base, K)])
        pltpu.sync_copy(aux_c, aux_sh.at[dst_v], add=True)
        return _

    lax.fori_loop(0, nch, chunk_body, None)
    plsc.subcore_barrier()

    rpt = (n // NS) // 8 * 8
    tail = n - NS * rpt
    row0 = sid * rpt
    off = cid * n + row0
    pltpu.sync_copy(aux_sh.at[pl.ds(row0, rpt)], aux_out.at[pl.ds(off, rpt)])
    if tail:
        @pl.when(sid == NS - 1)
        def _dump_tail():
            toff = cid * n + NS * rpt
            pltpu.sync_copy(aux_sh.at[pl.ds(NS * rpt, tail)],
                            aux_out.at[pl.ds(toff, tail)])


def _msg_body(nchunks_tot,
              xl_hbm, src_hbm, dst_hbm, ext_hbm,
              msg_out,
              src_v, dst_v, dstw_v, ex0_v, ex1_v, ex2_v, ex3_v, xlg, msg_c, zbuf,
              msg_sh, sem1, sem2):
    n = xl_hbm.shape[0]
    win_n = n // 2
    cid = lax.axis_index("c")
    sid = lax.axis_index("s")
    wid = sid * NC + cid
    iota = lax.iota(jnp.int32, L)
    zero16 = jnp.zeros((L,), jnp.float32)

    zr = zbuf.shape[0]
    for i in range(zr):
        for j in range(D // L):
            zbuf[i, pl.ds(j * L, L)] = zero16

    nch = (nchunks_tot // NW) + jnp.where(wid < (nchunks_tot % NW), 1, 0)
    wrows = win_n + TRASH

    for win in range(2):
        wbase = win * win_n

        zrows = wrows // NS
        zrow0 = sid * zrows
        nfull, rem = divmod(zrows, zr)
        for ci in range(nfull):
            pltpu.sync_copy(zbuf, msg_sh.at[pl.ds(zrow0 + ci * zr, zr)])
        if rem:
            pltpu.sync_copy(zbuf.at[pl.ds(0, rem)],
                            msg_sh.at[pl.ds(zrow0 + nfull * zr, rem)])
        plsc.subcore_barrier()

        def chunk_body(i, _):
            chunk = wid + i * NW
            base = chunk * K
            pltpu.sync_copy(src_hbm.at[pl.ds(base, K)], src_v)
            pltpu.sync_copy(dst_hbm.at[pl.ds(base, K)], dst_v)
            cp1 = pltpu.async_copy(xl_hbm.at[src_v], xlg, sem1)
            for h, exh in enumerate((ex0_v, ex1_v, ex2_v, ex3_v)):
                pltpu.sync_copy(ext_hbm.at[h, pl.ds(base, K)], exh)
            for j in range(K // L):
                dv = dst_v[pl.ds(j * L, L)]
                dw = dv - wbase
                inw = (dw >= 0) & (dw < win_n)
                dstw_v[pl.ds(j * L, L)] = jnp.where(inw, dw,
                                                    win_n + (dv & (TRASH - 1)))
            cp1.wait()

            def edge_body(e, _):
                se = jnp.full((L,), e, jnp.int32)
                exu = [plsc.load_gather(exh, [se])
                       for exh in (ex0_v, ex1_v, ex2_v, ex3_v)]
                for j in range(D // L):
                    msg_c[e, pl.ds(j * L, L)] = exu[j // 2] * xlg[e, pl.ds(j * L, L)]
                return _

            lax.fori_loop(0, K, edge_body, None, unroll=4)

            pltpu.sync_copy(msg_c, msg_sh.at[dstw_v], add=True)
            return _

        lax.fori_loop(0, nch, chunk_body, None)
        plsc.subcore_barrier()

        rpt = (win_n // NS) // 8 * 8
        dtail = win_n - NS * rpt
        drow0 = sid * rpt
        off = cid * n + wbase + drow0
        pltpu.sync_copy(msg_sh.at[pl.ds(drow0, rpt)], msg_out.at[pl.ds(off, rpt)])
        if dtail:
            @pl.when(sid == NS - 1)
            def _dump_tail():
                toff = cid * n + wbase + NS * rpt
                pltpu.sync_copy(msg_sh.at[pl.ds(NS * rpt, dtail)],
                                msg_out.at[pl.ds(toff, dtail)])
        plsc.subcore_barrier()


def _final_body(msg_ref, aux_ref, xl_ref, xr_ref, we_ref, matt_ref, mbh_ref,
                mb16_ref, bias_ref, out_ref):
    aux = aux_ref[0] + aux_ref[1]
    msg = msg_ref[0] + msg_ref[1]
    xl = xl_ref[...]
    deg = jnp.maximum(aux[:, 4:5], 1.0)
    la = aux[:, 5:8] / deg
    xe = jnp.dot(la, we_ref[...], preferred_element_type=jnp.float32)
    u = xl + xr_ref[...] + xe
    el = jnp.maximum(u, NEG * u)
    alpha_pad = jnp.dot(el, matt_ref[...], preferred_element_type=jnp.float32)
    exp_pad = jnp.exp(alpha_pad)
    ex_b = jnp.dot(exp_pad, mbh_ref[...], preferred_element_type=jnp.float32)
    den_b = jnp.dot(aux, mb16_ref[...], preferred_element_type=jnp.float32)
    out_ref[...] = (msg + ex_b * xl) / (den_b + ex_b + 1e-16) + bias_ref[...]


def kernel(x, edge_index, edge_attr, W_l, b_l, W_r, b_r, W_e, att, bias):
    n, d_in = x.shape
    e_num = edge_index.shape[1]
    assert e_num % K == 0 and n % NS == 0 and (n // 2) % 8 == 0

    src = edge_index[0]
    dst = edge_index[1]
    ea_flat = jnp.pad(edge_attr, ((0, 0), (0, 1))).reshape(-1)
    att_flat = att.reshape(-1)

    carange = jnp.arange(D)
    hsel = carange // C
    matt = jnp.zeros((D, D), jnp.float32).at[carange, hsel].set(att_flat)
    mbh = (jnp.arange(D)[:, None] == hsel[None, :]).astype(jnp.float32)
    mbh = mbh * (jnp.arange(D)[:, None] < HEADS)
    mb16 = (jnp.arange(16)[:, None] == hsel[None, :]).astype(jnp.float32)
    mb16 = mb16 * (jnp.arange(16)[:, None] < HEADS)

    bn = 2000
    xl, xr = pl.pallas_call(
        _proj_body,
        grid=(n // bn,),
        in_specs=[
            pl.BlockSpec((bn, d_in), lambda i: (i, 0)),
            pl.BlockSpec((d_in, D), lambda i: (0, 0)),
            pl.BlockSpec((D,), lambda i: (0,)),
            pl.BlockSpec((d_in, D), lambda i: (0, 0)),
            pl.BlockSpec((D,), lambda i: (0,)),
        ],
        out_specs=[
            pl.BlockSpec((bn, D), lambda i: (i, 0)),
            pl.BlockSpec((bn, D), lambda i: (i, 0)),
        ],
        out_shape=[
            jax.ShapeDtypeStruct((n, D), jnp.float32),
            jax.ShapeDtypeStruct((n, D), jnp.float32),
        ],
    )(x, W_l, b_l, W_r, b_r)

    be = 8000
    xe = pl.pallas_call(
        _xe_body,
        grid=(e_num // be,),
        in_specs=[
            pl.BlockSpec((be, 3), lambda i: (i, 0)),
            pl.BlockSpec((3, D), lambda i: (0, 0)),
        ],
        out_specs=pl.BlockSpec((be, D), lambda i: (i, 0)),
        out_shape=jax.ShapeDtypeStruct((e_num, D), jnp.float32),
    )(edge_attr, W_e)

    nchunks = e_num // K
    mesh = plsc.VectorSubcoreMesh(core_axis_name="c", subcore_axis_name="s",
                                  num_cores=NC, num_subcores=NS)
    sc_params = pltpu.CompilerParams(needs_layout_passes=False,
                                     use_tc_tiling_on_sc=False)

    alpha_call = pl.kernel(
        functools.partial(_alpha_body, nchunks),
        out_type=[
            jax.ShapeDtypeStruct((NC * n, 16), jnp.float32),
            jax.ShapeDtypeStruct((e_num, 16), jnp.float32),
        ],
        mesh=mesh,
        compiler_params=sc_params,
        scratch_types=[
            pltpu.VMEM((D,), jnp.float32),
            pltpu.VMEM((K,), jnp.int32),
            pltpu.VMEM((K,), jnp.int32),
            pltpu.VMEM((K * 4 + L,), jnp.float32),
            pltpu.VMEM((K, D), jnp.float32),
            pltpu.VMEM((K, D), jnp.float32),
            pltpu.VMEM((K, D), jnp.float32),
            pltpu.VMEM((K, 16), jnp.float32),
            pltpu.VMEM((4 * L,), jnp.float32),
            pltpu.VMEM((4 * L,), jnp.float32),
            pltpu.VMEM((L,), jnp.float32),
            pltpu.VMEM((50, 16), jnp.float32),
            pltpu.VMEM_SHARED((n, 16), jnp.float32),
            pltpu.SemaphoreType.DMA,
            pltpu.SemaphoreType.DMA,
            pltpu.SemaphoreType.DMA,
        ],
    )
    aux_p, exrows = alpha_call(xl, xr, xe, src, dst, ea_flat, att_flat)
    aux_p = aux_p.reshape(NC, n, 16)

    msg_call = pl.kernel(
        functools.partial(_msg_body, nchunks),
        out_type=jax.ShapeDtypeStruct((NC * n, D), jnp.float32),
        mesh=mesh,
        compiler_params=sc_params,
        scratch_types=[
            pltpu.VMEM((K,), jnp.int32),
            pltpu.VMEM((K,), jnp.int32),
            pltpu.VMEM((K,), jnp.int32),
            pltpu.VMEM((K,), jnp.float32),
            pltpu.VMEM((K,), jnp.float32),
            pltpu.VMEM((K,), jnp.float32),
            pltpu.VMEM((K,), jnp.float32),
            pltpu.VMEM((K, D), jnp.float32),
            pltpu.VMEM((K, D), jnp.float32),
            pltpu.VMEM((48, D), jnp.float32),
            pltpu.VMEM_SHARED((n // 2 + TRASH, D), jnp.float32),
            pltpu.SemaphoreType.DMA,
            pltpu.SemaphoreType.DMA,
        ],
    )
    ext = jnp.transpose(exrows[:, :HEADS])
    msg_p = msg_call(xl, src, dst, ext).reshape(NC, n, D)

    out = pl.pallas_call(
        _final_body,
        grid=(n // bn,),
        in_specs=[
            pl.BlockSpec((NC, bn, D), lambda i: (0, i, 0)),
            pl.BlockSpec((NC, bn, 16), lambda i: (0, i, 0)),
            pl.BlockSpec((bn, D), lambda i: (i, 0)),
            pl.BlockSpec((bn, D), lambda i: (i, 0)),
            pl.BlockSpec((3, D), lambda i: (0, 0)),
            pl.BlockSpec((D, D), lambda i: (0, 0)),
            pl.BlockSpec((D, D), lambda i: (0, 0)),
            pl.BlockSpec((16, D), lambda i: (0, 0)),
            pl.BlockSpec((D,), lambda i: (0,)),
        ],
        out_specs=pl.BlockSpec((bn, D), lambda i: (i, 0)),
        out_shape=jax.ShapeDtypeStruct((n, D), jnp.float32),
    )(msg_p, aux_p, xl, xr, W_e, matt, mbh, mb16, bias)

    return out


# speedup vs baseline: 1.1120x; 1.1120x over previous
import functools
import jax
import jax.numpy as jnp
from jax import lax
from jax.experimental import pallas as pl
from jax.experimental.pallas import tpu as pltpu
from jax.experimental.pallas import tpu_sc as plsc

HEADS = 4
C = 32
D = HEADS * C
NEG = 0.2
K = 128
L = 16
NC = 2
NS = 16
NW = NC * NS
TRASH = 8


def _proj_body(x_ref, wl_ref, bl_ref, wr_ref, br_ref, xl_ref, xr_ref):
    x = x_ref[...]
    xl_ref[...] = jnp.dot(x, wl_ref[...], preferred_element_type=jnp.float32) + bl_ref[...]
    xr_ref[...] = jnp.dot(x, wr_ref[...], preferred_element_type=jnp.float32) + br_ref[...]


def _xe_body(ea_ref, we_ref, xe_ref):
    xe_ref[...] = jnp.dot(ea_ref[...], we_ref[...], preferred_element_type=jnp.float32)


def _alpha_body(nchunks_tot,
                xl_hbm, xr_hbm, xe_hbm, src_hbm, dst_hbm, ea_hbm, att_hbm,
                aux_out, exr_out,
                attv, src_v, dst_v, ea_v, xlg, xrg, xeg, aux_c,
                abuf, rbuf, ebuf, zbufa, aux_sh, sem1, sem2, sem3):
    n = xl_hbm.shape[0]
    pltpu.sync_copy(att_hbm, attv)
    cid = lax.axis_index("c")
    sid = lax.axis_index("s")
    wid = sid * NC + cid
    iota = lax.iota(jnp.int32, L)
    iota4 = iota * 4
    zero16 = jnp.zeros((L,), jnp.float32)

    zr = zbufa.shape[0]
    for i in range(zr):
        zbufa[i, :] = zero16

    att_r = [attv[pl.ds(j * L, L)] for j in range(D // L)]

    for j in range(1, 4):
        rbuf[pl.ds(j * L, L)] = zero16

    zrows = n // NS
    zrow0 = sid * zrows
    nfull, rem = divmod(zrows, zr)
    for ci in range(nfull):
        pltpu.sync_copy(zbufa, aux_sh.at[pl.ds(zrow0 + ci * zr, zr)])
    if rem:
        pltpu.sync_copy(zbufa.at[pl.ds(0, rem)],
                        aux_sh.at[pl.ds(zrow0 + nfull * zr, rem)])
    plsc.subcore_barrier()

    nch = (nchunks_tot // NW) + jnp.where(wid < (nchunks_tot % NW), 1, 0)

    def chunk_body(i, _):
        chunk = wid + i * NW
        base = chunk * K
        ci1 = pltpu.async_copy(src_hbm.at[pl.ds(base, K)], src_v, sem1)
        ci2 = pltpu.async_copy(dst_hbm.at[pl.ds(base, K)], dst_v, sem1)
        ci3 = pltpu.async_copy(ea_hbm.at[pl.ds(base * 4, K * 4)],
                               ea_v.at[pl.ds(0, K * 4)], sem1)
        cp3 = pltpu.async_copy(xe_hbm.at[pl.ds(base, K), :], xeg, sem3)
        ci1.wait()
        ci2.wait()
        ci3.wait()
        cp1 = pltpu.async_copy(xl_hbm.at[src_v], xlg, sem1)
        cp2 = pltpu.async_copy(xr_hbm.at[dst_v], xrg, sem2)
        cp1.wait()
        cp2.wait()
        cp3.wait()

        def edge_body(e, _):
            xlr = [xlg[e, pl.ds(j * L, L)] for j in range(D // L)]
            xrr = [xrg[e, pl.ds(j * L, L)] for j in range(D // L)]
            xer = [xeg[e, pl.ds(j * L, L)] for j in range(D // L)]
            for h in range(HEADS):
                t = zero16
                for j in (2 * h, 2 * h + 1):
                    u = xlr[j] + xrr[j] + xer[j]
                    lr = jnp.maximum(u, NEG * u)
                    t = t + lr * att_r[j]
                abuf[pl.ds(h * L, L)] = t
            s = zero16
            for rr in range(4):
                s = s + plsc.load_gather(abuf, [iota4 + rr])
            rbuf[pl.ds(0, L)] = s
            w = zero16
            for rr in range(4):
                w = w + plsc.load_gather(rbuf, [iota4 + rr])
            expv = jnp.exp(w)
            eav = plsc.load_gather(ea_v, [4 * e + jnp.maximum(iota - 5, 0)])
            r = jnp.where(iota < 5, expv, jnp.where(iota < 8, eav, zero16))
            aux_c[e, :] = r
            return _

        lax.fori_loop(0, K, edge_body, None, unroll=4)

        pltpu.sync_copy(aux_c, exr_out.at[pl.ds(base, K)])
        pltpu.sync_copy(aux_c, aux_sh.at[dst_v], add=True)
        return _

    lax.fori_loop(0, nch, chunk_body, None)
    plsc.subcore_barrier()

    rpt = (n // NS) // 8 * 8
    tail = n - NS * rpt
    row0 = sid * rpt
    off = cid * n + row0
    pltpu.sync_copy(aux_sh.at[pl.ds(row0, rpt)], aux_out.at[pl.ds(off, rpt)])
    if tail:
        @pl.when(sid == NS - 1)
        def _dump_tail():
            toff = cid * n + NS * rpt
            pltpu.sync_copy(aux_sh.at[pl.ds(NS * rpt, tail)],
                            aux_out.at[pl.ds(toff, tail)])


def _msg_body(nchunks_tot,
              xl_hbm, src_hbm, dst_hbm, ext_hbm,
              msg_out,
              src_v, dst_v, dstw_v, ex0_v, ex1_v, ex2_v, ex3_v, xlg, msg_c, zbuf,
              msg_sh, sem1, sem2):
    n = xl_hbm.shape[0]
    win_n = n // 2
    cid = lax.axis_index("c")
    sid = lax.axis_index("s")
    wid = sid * NC + cid
    iota = lax.iota(jnp.int32, L)
    zero16 = jnp.zeros((L,), jnp.float32)

    zr = zbuf.shape[0]
    for i in range(zr):
        for j in range(D // L):
            zbuf[i, pl.ds(j * L, L)] = zero16

    nch = (nchunks_tot // NW) + jnp.where(wid < (nchunks_tot % NW), 1, 0)
    wrows = win_n + TRASH

    for win in range(2):
        wbase = win * win_n

        zrows = wrows // NS
        zrow0 = sid * zrows
        nfull, rem = divmod(zrows, zr)
        for ci in range(nfull):
            pltpu.sync_copy(zbuf, msg_sh.at[pl.ds(zrow0 + ci * zr, zr)])
        if rem:
            pltpu.sync_copy(zbuf.at[pl.ds(0, rem)],
                            msg_sh.at[pl.ds(zrow0 + nfull * zr, rem)])
        plsc.subcore_barrier()

        def chunk_body(i, _):
            chunk = wid + i * NW
            base = chunk * K
            ci1 = pltpu.async_copy(src_hbm.at[pl.ds(base, K)], src_v, sem1)
            ci2 = pltpu.async_copy(dst_hbm.at[pl.ds(base, K)], dst_v, sem1)
            cpe = [pltpu.async_copy(ext_hbm.at[h, pl.ds(base, K)], exh, sem2)
                   for h, exh in enumerate((ex0_v, ex1_v, ex2_v, ex3_v))]
            ci1.wait()
            ci2.wait()
            cp1 = pltpu.async_copy(xl_hbm.at[src_v], xlg, sem1)
            for j in range(K // L):
                dv = dst_v[pl.ds(j * L, L)]
                dw = dv - wbase
                inw = (dw >= 0) & (dw < win_n)
                dstw_v[pl.ds(j * L, L)] = jnp.where(inw, dw,
                                                    win_n + (dv & (TRASH - 1)))
            cp1.wait()
            for cp in cpe:
                cp.wait()

            def edge_body(e, _):
                se = jnp.full((L,), e, jnp.int32)
                exu = [plsc.load_gather(exh, [se])
                       for exh in (ex0_v, ex1_v, ex2_v, ex3_v)]
                for j in range(D // L):
                    msg_c[e, pl.ds(j * L, L)] = exu[j // 2] * xlg[e, pl.ds(j * L, L)]
                return _

            lax.fori_loop(0, K, edge_body, None, unroll=4)

            pltpu.sync_copy(msg_c, msg_sh.at[dstw_v], add=True)
            return _

        lax.fori_loop(0, nch, chunk_body, None)
        plsc.subcore_barrier()

        rpt = (win_n // NS) // 8 * 8
        dtail = win_n - NS * rpt
        drow0 = sid * rpt
        off = cid * n + wbase + drow0
        pltpu.sync_copy(msg_sh.at[pl.ds(drow0, rpt)], msg_out.at[pl.ds(off, rpt)])
        if dtail:
            @pl.when(sid == NS - 1)
            def _dump_tail():
                toff = cid * n + wbase + NS * rpt
                pltpu.sync_copy(msg_sh.at[pl.ds(NS * rpt, dtail)],
                                msg_out.at[pl.ds(toff, dtail)])
        plsc.subcore_barrier()


def _final_body(msg_ref, aux_ref, xl_ref, xr_ref, we_ref, matt_ref, mbh_ref,
                mb16_ref, bias_ref, out_ref):
    aux = aux_ref[0] + aux_ref[1]
    msg = msg_ref[0] + msg_ref[1]
    xl = xl_ref[...]
    deg = jnp.maximum(aux[:, 4:5], 1.0)
    la = aux[:, 5:8] / deg
    xe = jnp.dot(la, we_ref[...], preferred_element_type=jnp.float32)
    u = xl + xr_ref[...] + xe
    el = jnp.maximum(u, NEG * u)
    alpha_pad = jnp.dot(el, matt_ref[...], preferred_element_type=jnp.float32)
    exp_pad = jnp.exp(alpha_pad)
    ex_b = jnp.dot(exp_pad, mbh_ref[...], preferred_element_type=jnp.float32)
    den_b = jnp.dot(aux, mb16_ref[...], preferred_element_type=jnp.float32)
    out_ref[...] = (msg + ex_b * xl) / (den_b + ex_b + 1e-16) + bias_ref[...]


def kernel(x, edge_index, edge_attr, W_l, b_l, W_r, b_r, W_e, att, bias):
    n, d_in = x.shape
    e_num = edge_index.shape[1]
    assert e_num % K == 0 and n % NS == 0 and (n // 2) % 8 == 0

    src = edge_index[0]
    dst = edge_index[1]
    ea_flat = jnp.pad(edge_attr, ((0, 0), (0, 1))).reshape(-1)
    att_flat = att.reshape(-1)

    carange = jnp.arange(D)
    hsel = carange // C
    matt = jnp.zeros((D, D), jnp.float32).at[carange, hsel].set(att_flat)
    mbh = (jnp.arange(D)[:, None] == hsel[None, :]).astype(jnp.float32)
    mbh = mbh * (jnp.arange(D)[:, None] < HEADS)
    mb16 = (jnp.arange(16)[:, None] == hsel[None, :]).astype(jnp.float32)
    mb16 = mb16 * (jnp.arange(16)[:, None] < HEADS)

    bn = 2000
    xl, xr = pl.pallas_call(
        _proj_body,
        grid=(n // bn,),
        in_specs=[
            pl.BlockSpec((bn, d_in), lambda i: (i, 0)),
            pl.BlockSpec((d_in, D), lambda i: (0, 0)),
            pl.BlockSpec((D,), lambda i: (0,)),
            pl.BlockSpec((d_in, D), lambda i: (0, 0)),
            pl.BlockSpec((D,), lambda i: (0,)),
        ],
        out_specs=[
            pl.BlockSpec((bn, D), lambda i: (i, 0)),
            pl.BlockSpec((bn, D), lambda i: (i, 0)),
        ],
        out_shape=[
            jax.ShapeDtypeStruct((n, D), jnp.float32),
            jax.ShapeDtypeStruct((n, D), jnp.float32),
        ],
    )(x, W_l, b_l, W_r, b_r)

    be = 8000
    xe = pl.pallas_call(
        _xe_body,
        grid=(e_num // be,),
        in_specs=[
            pl.BlockSpec((be, 3), lambda i: (i, 0)),
            pl.BlockSpec((3, D), lambda i: (0, 0)),
        ],
        out_specs=pl.BlockSpec((be, D), lambda i: (i, 0)),
        out_shape=jax.ShapeDtypeStruct((e_num, D), jnp.float32),
    )(edge_attr, W_e)

    nchunks = e_num // K
    mesh = plsc.VectorSubcoreMesh(core_axis_name="c", subcore_axis_name="s",
                                  num_cores=NC, num_subcores=NS)
    sc_params = pltpu.CompilerParams(needs_layout_passes=False,
                                     use_tc_tiling_on_sc=False)

    alpha_call = pl.kernel(
        functools.partial(_alpha_body, nchunks),
        out_type=[
            jax.ShapeDtypeStruct((NC * n, 16), jnp.float32),
            jax.ShapeDtypeStruct((e_num, 16), jnp.float32),
        ],
        mesh=mesh,
        compiler_params=sc_params,
        scratch_types=[
            pltpu.VMEM((D,), jnp.float32),
            pltpu.VMEM((K,), jnp.int32),
            pltpu.VMEM((K,), jnp.int32),
            pltpu.VMEM((K * 4 + L,), jnp.float32),
            pltpu.VMEM((K, D), jnp.float32),
            pltpu.VMEM((K, D), jnp.float32),
            pltpu.VMEM((K, D), jnp.float32),
            pltpu.VMEM((K, 16), jnp.float32),
            pltpu.VMEM((4 * L,), jnp.float32),
            pltpu.VMEM((4 * L,), jnp.float32),
            pltpu.VMEM((L,), jnp.float32),
            pltpu.VMEM((50, 16), jnp.float32),
            pltpu.VMEM_SHARED((n, 16), jnp.float32),
            pltpu.SemaphoreType.DMA,
            pltpu.SemaphoreType.DMA,
            pltpu.SemaphoreType.DMA,
        ],
    )
    aux_p, exrows = alpha_call(xl, xr, xe, src, dst, ea_flat, att_flat)
    aux_p = aux_p.reshape(NC, n, 16)

    msg_call = pl.kernel(
        functools.partial(_msg_body, nchunks),
        out_type=jax.ShapeDtypeStruct((NC * n, D), jnp.float32),
        mesh=mesh,
        compiler_params=sc_params,
        scratch_types=[
            pltpu.VMEM((K,), jnp.int32),
            pltpu.VMEM((K,), jnp.int32),
            pltpu.VMEM((K,), jnp.int32),
            pltpu.VMEM((K,), jnp.float32),
            pltpu.VMEM((K,), jnp.float32),
            pltpu.VMEM((K,), jnp.float32),
            pltpu.VMEM((K,), jnp.float32),
            pltpu.VMEM((K, D), jnp.float32),
            pltpu.VMEM((K, D), jnp.float32),
            pltpu.VMEM((48, D), jnp.float32),
            pltpu.VMEM_SHARED((n // 2 + TRASH, D), jnp.float32),
            pltpu.SemaphoreType.DMA,
            pltpu.SemaphoreType.DMA,
        ],
    )
    ext = jnp.transpose(exrows[:, :HEADS])
    msg_p = msg_call(xl, src, dst, ext).reshape(NC, n, D)

    out = pl.pallas_call(
        _final_body,
        grid=(n // bn,),
        in_specs=[
            pl.BlockSpec((NC, bn, D), lambda i: (0, i, 0)),
            pl.BlockSpec((NC, bn, 16), lambda i: (0, i, 0)),
            pl.BlockSpec((bn, D), lambda i: (i, 0)),
            pl.BlockSpec((bn, D), lambda i: (i, 0)),
            pl.BlockSpec((3, D), lambda i: (0, 0)),
            pl.BlockSpec((D, D), lambda i: (0, 0)),
            pl.BlockSpec((D, D), lambda i: (0, 0)),
            pl.BlockSpec((16, D), lambda i: (0, 0)),
            pl.BlockSpec((D,), lambda i: (0,)),
        ],
        out_specs=pl.BlockSpec((bn, D), lambda i: (i, 0)),
        out_shape=jax.ShapeDtypeStruct((n, D), jnp.float32),
    )(msg_p, aux_p, xl, xr, W_e, matt, mbh, mb16, bias)

    return out


# speedup vs baseline: 1.1460x; 1.0305x over previous
import functools
import jax
import jax.numpy as jnp
from jax import lax
from jax.experimental import pallas as pl
from jax.experimental.pallas import tpu as pltpu
from jax.experimental.pallas import tpu_sc as plsc

HEADS = 4
C = 32
D = HEADS * C
NEG = 0.2
K = 128
L = 16
NC = 2
NS = 16
NW = NC * NS
TRASH = 8


def _proj_body(x_ref, wl_ref, bl_ref, wr_ref, br_ref, xl_ref, xr_ref):
    x = x_ref[...]
    xl_ref[...] = jnp.dot(x, wl_ref[...], preferred_element_type=jnp.float32) + bl_ref[...]
    xr_ref[...] = jnp.dot(x, wr_ref[...], preferred_element_type=jnp.float32) + br_ref[...]


def _xe_body(ea_ref, we_ref, xe_ref):
    xe_ref[...] = jnp.dot(ea_ref[...], we_ref[...], preferred_element_type=jnp.float32)


def _alpha_body(nchunks_tot,
                xl_hbm, xr_hbm, xe_hbm, src_hbm, dst_hbm, ea_hbm, att_hbm,
                aux_out, exr_out,
                attv, src_v, dst_v, ea_v, xlg, xrg, xeg, aux_c,
                abuf, rbuf, ebuf, zbufa, aux_sh, sem1, sem2, sem3):
    n = xl_hbm.shape[0]
    pltpu.sync_copy(att_hbm, attv)
    cid = lax.axis_index("c")
    sid = lax.axis_index("s")
    wid = sid * NC + cid
    iota = lax.iota(jnp.int32, L)
    iota4 = iota * 4
    zero16 = jnp.zeros((L,), jnp.float32)

    zr = zbufa.shape[0]
    for i in range(zr):
        zbufa[i, :] = zero16

    att_r = [attv[pl.ds(j * L, L)] for j in range(D // L)]

    for j in range(1, 4):
        rbuf[pl.ds(j * L, L)] = zero16

    zrows = n // NS
    zrow0 = sid * zrows
    nfull, rem = divmod(zrows, zr)
    for ci in range(nfull):
        pltpu.sync_copy(zbufa, aux_sh.at[pl.ds(zrow0 + ci * zr, zr)])
    if rem:
        pltpu.sync_copy(zbufa.at[pl.ds(0, rem)],
                        aux_sh.at[pl.ds(zrow0 + nfull * zr, rem)])
    plsc.subcore_barrier()

    nch = (nchunks_tot // NW) + jnp.where(wid < (nchunks_tot % NW), 1, 0)

    def chunk_body(i, _):
        chunk = wid + i * NW
        base = chunk * K
        ci1 = pltpu.async_copy(src_hbm.at[pl.ds(base, K)], src_v, sem1)
        ci2 = pltpu.async_copy(dst_hbm.at[pl.ds(base, K)], dst_v, sem1)
        ci3 = pltpu.async_copy(ea_hbm.at[pl.ds(base * 4, K * 4)],
                               ea_v.at[pl.ds(0, K * 4)], sem1)
        cp3 = pltpu.async_copy(xe_hbm.at[pl.ds(base, K), :], xeg, sem3)
        ci1.wait()
        ci2.wait()
        ci3.wait()
        cp1 = pltpu.async_copy(xl_hbm.at[src_v], xlg, sem1)
        cp2 = pltpu.async_copy(xr_hbm.at[dst_v], xrg, sem2)
        cp1.wait()
        cp2.wait()
        cp3.wait()

        def edge_body(e, _):
            xlr = [xlg[e, pl.ds(j * L, L)] for j in range(D // L)]
            xrr = [xrg[e, pl.ds(j * L, L)] for j in range(D // L)]
            xer = [xeg[e, pl.ds(j * L, L)] for j in range(D // L)]
            for h in range(HEADS):
                t = zero16
                for j in (2 * h, 2 * h + 1):
                    u = xlr[j] + xrr[j] + xer[j]
                    lr = jnp.maximum(u, NEG * u)
                    t = t + lr * att_r[j]
                abuf[pl.ds(h * L, L)] = t
            s = zero16
            for rr in range(4):
                s = s + plsc.load_gather(abuf, [iota4 + rr])
            rbuf[pl.ds(0, L)] = s
            w = zero16
            for rr in range(4):
                w = w + plsc.load_gather(rbuf, [iota4 + rr])
            expv = jnp.exp(w)
            eav = plsc.load_gather(ea_v, [4 * e + jnp.maximum(iota - 5, 0)])
            r = jnp.where(iota < 5, expv, jnp.where(iota < 8, eav, zero16))
            aux_c[e, :] = r
            return _

        lax.fori_loop(0, K, edge_body, None, unroll=4)

        pltpu.sync_copy(aux_c, exr_out.at[pl.ds(base, K)])
        pltpu.sync_copy(aux_c, aux_sh.at[dst_v], add=True)
        return _

    lax.fori_loop(0, nch, chunk_body, None)
    plsc.subcore_barrier()

    rpt = (n // NS) // 8 * 8
    tail = n - NS * rpt
    row0 = sid * rpt
    off = cid * n + row0
    pltpu.sync_copy(aux_sh.at[pl.ds(row0, rpt)], aux_out.at[pl.ds(off, rpt)])
    if tail:
        @pl.when(sid == NS - 1)
        def _dump_tail():
            toff = cid * n + NS * rpt
            pltpu.sync_copy(aux_sh.at[pl.ds(NS * rpt, tail)],
                            aux_out.at[pl.ds(toff, tail)])


def _msg_body(nchunks_tot,
              xl_hbm, src_hbm, dst_hbm, ext_hbm,
              msg_out,
              src_v, dst_v, dstw_v, ex0_v, ex1_v, ex2_v, ex3_v, xlg, msg_c, zbuf,
              msg_sh, sem1, sem2, semsc):
    n = xl_hbm.shape[0]
    win_n = n // 2
    cid = lax.axis_index("c")
    sid = lax.axis_index("s")
    wid = sid * NC + cid
    iota = lax.iota(jnp.int32, L)
    zero16 = jnp.zeros((L,), jnp.float32)

    zr = zbuf.shape[0]
    for i in range(zr):
        for j in range(D // L):
            zbuf[i, pl.ds(j * L, L)] = zero16

    nch = (nchunks_tot // NW) + jnp.where(wid < (nchunks_tot % NW), 1, 0)
    wrows = win_n + TRASH

    for win in range(2):
        wbase = win * win_n

        zrows = wrows // NS
        zrow0 = sid * zrows
        nfull, rem = divmod(zrows, zr)
        for ci in range(nfull):
            pltpu.sync_copy(zbuf, msg_sh.at[pl.ds(zrow0 + ci * zr, zr)])
        if rem:
            pltpu.sync_copy(zbuf.at[pl.ds(0, rem)],
                            msg_sh.at[pl.ds(zrow0 + nfull * zr, rem)])
        plsc.subcore_barrier()

        def chunk_body(i, _):
            chunk = wid + i * NW
            base = chunk * K
            ci1 = pltpu.async_copy(src_hbm.at[pl.ds(base, K)], src_v, sem1)
            ci2 = pltpu.async_copy(dst_hbm.at[pl.ds(base, K)], dst_v, sem1)
            cpe = [pltpu.async_copy(ext_hbm.at[h, pl.ds(base, K)], exh, sem2)
                   for h, exh in enumerate((ex0_v, ex1_v, ex2_v, ex3_v))]

            @pl.when(i > 0)
            def _drain_prev():
                pltpu.make_async_copy(xl_hbm.at[pl.ds(0, K), :], msg_c,
                                      semsc).wait()

            ci1.wait()
            ci2.wait()
            cp1 = pltpu.async_copy(xl_hbm.at[src_v], xlg, sem1)
            for j in range(K // L):
                dv = dst_v[pl.ds(j * L, L)]
                dw = dv - wbase
                inw = (dw >= 0) & (dw < win_n)
                dstw_v[pl.ds(j * L, L)] = jnp.where(inw, dw,
                                                    win_n + (dv & (TRASH - 1)))
            cp1.wait()
            for cp in cpe:
                cp.wait()

            def edge_body(e, _):
                se = jnp.full((L,), e, jnp.int32)
                exu = [plsc.load_gather(exh, [se])
                       for exh in (ex0_v, ex1_v, ex2_v, ex3_v)]
                for j in range(D // L):
                    msg_c[e, pl.ds(j * L, L)] = exu[j // 2] * xlg[e, pl.ds(j * L, L)]
                return _

            lax.fori_loop(0, K, edge_body, None, unroll=4)

            pltpu.async_copy(msg_c, msg_sh.at[dstw_v], semsc, add=True)
            return _

        lax.fori_loop(0, nch, chunk_body, None)

        @pl.when(nch > 0)
        def _drain_last():
            pltpu.make_async_copy(xl_hbm.at[pl.ds(0, K), :], msg_c,
                                  semsc).wait()

        plsc.subcore_barrier()

        rpt = (win_n // NS) // 8 * 8
        dtail = win_n - NS * rpt
        drow0 = sid * rpt
        off = cid * n + wbase + drow0
        pltpu.sync_copy(msg_sh.at[pl.ds(drow0, rpt)], msg_out.at[pl.ds(off, rpt)])
        if dtail:
            @pl.when(sid == NS - 1)
            def _dump_tail():
                toff = cid * n + wbase + NS * rpt
                pltpu.sync_copy(msg_sh.at[pl.ds(NS * rpt, dtail)],
                                msg_out.at[pl.ds(toff, dtail)])
        plsc.subcore_barrier()


def _final_body(msg_ref, aux_ref, xl_ref, xr_ref, we_ref, matt_ref, mbh_ref,
                mb16_ref, bias_ref, out_ref):
    aux = aux_ref[0] + aux_ref[1]
    msg = msg_ref[0] + msg_ref[1]
    xl = xl_ref[...]
    deg = jnp.maximum(aux[:, 4:5], 1.0)
    la = aux[:, 5:8] / deg
    xe = jnp.dot(la, we_ref[...], preferred_element_type=jnp.float32)
    u = xl + xr_ref[...] + xe
    el = jnp.maximum(u, NEG * u)
    alpha_pad = jnp.dot(el, matt_ref[...], preferred_element_type=jnp.float32)
    exp_pad = jnp.exp(alpha_pad)
    ex_b = jnp.dot(exp_pad, mbh_ref[...], preferred_element_type=jnp.float32)
    den_b = jnp.dot(aux, mb16_ref[...], preferred_element_type=jnp.float32)
    out_ref[...] = (msg + ex_b * xl) / (den_b + ex_b + 1e-16) + bias_ref[...]


def kernel(x, edge_index, edge_attr, W_l, b_l, W_r, b_r, W_e, att, bias):
    n, d_in = x.shape
    e_num = edge_index.shape[1]
    assert e_num % K == 0 and n % NS == 0 and (n // 2) % 8 == 0

    src = edge_index[0]
    dst = edge_index[1]
    ea_flat = jnp.pad(edge_attr, ((0, 0), (0, 1))).reshape(-1)
    att_flat = att.reshape(-1)

    carange = jnp.arange(D)
    hsel = carange // C
    matt = jnp.zeros((D, D), jnp.float32).at[carange, hsel].set(att_flat)
    mbh = (jnp.arange(D)[:, None] == hsel[None, :]).astype(jnp.float32)
    mbh = mbh * (jnp.arange(D)[:, None] < HEADS)
    mb16 = (jnp.arange(16)[:, None] == hsel[None, :]).astype(jnp.float32)
    mb16 = mb16 * (jnp.arange(16)[:, None] < HEADS)

    bn = 2000
    xl, xr = pl.pallas_call(
        _proj_body,
        grid=(n // bn,),
        in_specs=[
            pl.BlockSpec((bn, d_in), lambda i: (i, 0)),
            pl.BlockSpec((d_in, D), lambda i: (0, 0)),
            pl.BlockSpec((D,), lambda i: (0,)),
            pl.BlockSpec((d_in, D), lambda i: (0, 0)),
            pl.BlockSpec((D,), lambda i: (0,)),
        ],
        out_specs=[
            pl.BlockSpec((bn, D), lambda i: (i, 0)),
            pl.BlockSpec((bn, D), lambda i: (i, 0)),
        ],
        out_shape=[
            jax.ShapeDtypeStruct((n, D), jnp.float32),
            jax.ShapeDtypeStruct((n, D), jnp.float32),
        ],
    )(x, W_l, b_l, W_r, b_r)

    be = 8000
    xe = pl.pallas_call(
        _xe_body,
        grid=(e_num // be,),
        in_specs=[
            pl.BlockSpec((be, 3), lambda i: (i, 0)),
            pl.BlockSpec((3, D), lambda i: (0, 0)),
        ],
        out_specs=pl.BlockSpec((be, D), lambda i: (i, 0)),
        out_shape=jax.ShapeDtypeStruct((e_num, D), jnp.float32),
    )(edge_attr, W_e)

    nchunks = e_num // K
    mesh = plsc.VectorSubcoreMesh(core_axis_name="c", subcore_axis_name="s",
                                  num_cores=NC, num_subcores=NS)
    sc_params = pltpu.CompilerParams(needs_layout_passes=False,
                                     use_tc_tiling_on_sc=False)

    alpha_call = pl.kernel(
        functools.partial(_alpha_body, nchunks),
        out_type=[
            jax.ShapeDtypeStruct((NC * n, 16), jnp.float32),
            jax.ShapeDtypeStruct((e_num, 16), jnp.float32),
        ],
        mesh=mesh,
        compiler_params=sc_params,
        scratch_types=[
            pltpu.VMEM((D,), jnp.float32),
            pltpu.VMEM((K,), jnp.int32),
            pltpu.VMEM((K,), jnp.int32),
            pltpu.VMEM((K * 4 + L,), jnp.float32),
            pltpu.VMEM((K, D), jnp.float32),
            pltpu.VMEM((K, D), jnp.float32),
            pltpu.VMEM((K, D), jnp.float32),
            pltpu.VMEM((K, 16), jnp.float32),
            pltpu.VMEM((4 * L,), jnp.float32),
            pltpu.VMEM((4 * L,), jnp.float32),
            pltpu.VMEM((L,), jnp.float32),
            pltpu.VMEM((50, 16), jnp.float32),
            pltpu.VMEM_SHARED((n, 16), jnp.float32),
            pltpu.SemaphoreType.DMA,
            pltpu.SemaphoreType.DMA,
            pltpu.SemaphoreType.DMA,
        ],
    )
    aux_p, exrows = alpha_call(xl, xr, xe, src, dst, ea_flat, att_flat)
    aux_p = aux_p.reshape(NC, n, 16)

    msg_call = pl.kernel(
        functools.partial(_msg_body, nchunks),
        out_type=jax.ShapeDtypeStruct((NC * n, D), jnp.float32),
        mesh=mesh,
        compiler_params=sc_params,
        scratch_types=[
            pltpu.VMEM((K,), jnp.int32),
            pltpu.VMEM((K,), jnp.int32),
            pltpu.VMEM((K,), jnp.int32),
            pltpu.VMEM((K,), jnp.float32),
            pltpu.VMEM((K,), jnp.float32),
            pltpu.VMEM((K,), jnp.float32),
            pltpu.VMEM((K,), jnp.float32),
            pltpu.VMEM((K, D), jnp.float32),
            pltpu.VMEM((K, D), jnp.float32),
            pltpu.VMEM((48, D), jnp.float32),
            pltpu.VMEM_SHARED((n // 2 + TRASH, D), jnp.float32),
            pltpu.SemaphoreType.DMA,
            pltpu.SemaphoreType.DMA,
            pltpu.SemaphoreType.DMA,
        ],
    )
    ext = jnp.transpose(exrows[:, :HEADS])
    msg_p = msg_call(xl, src, dst, ext).reshape(NC, n, D)

    out = pl.pallas_call(
        _final_body,
        grid=(n // bn,),
        in_specs=[
            pl.BlockSpec((NC, bn, D), lambda i: (0, i, 0)),
            pl.BlockSpec((NC, bn, 16), lambda i: (0, i, 0)),
            pl.BlockSpec((bn, D), lambda i: (i, 0)),
            pl.BlockSpec((bn, D), lambda i: (i, 0)),
            pl.BlockSpec((3, D), lambda i: (0, 0)),
            pl.BlockSpec((D, D), lambda i: (0, 0)),
            pl.BlockSpec((D, D), lambda i: (0, 0)),
            pl.BlockSpec((16, D), lambda i: (0, 0)),
            pl.BlockSpec((D,), lambda i: (0,)),
        ],
        out_specs=pl.BlockSpec((bn, D), lambda i: (i, 0)),
        out_shape=jax.ShapeDtypeStruct((n, D), jnp.float32),
    )(msg_p, aux_p, xl, xr, W_e, matt, mbh, mb16, bias)

    return out
